# baseline (speedup 1.0000x reference)
import functools

import jax
import jax.numpy as jnp
import numpy as np
from jax import lax
from jax.experimental import pallas as pl
from jax.experimental.pallas import tpu as pltpu
from jax.experimental.pallas import tpu_sc as plsc

_MAX_LEN = 512
_LANE = 128


def _select_body(n_cores, n_planes, plane_w, out_w, per_sc, chunks, in_hbm,
                 g_hbm, out_hbm, stage_v, gidx_v, out_stage_v):
    wid = lax.axis_index("s") * n_cores + lax.axis_index("c")
    n_workers = n_cores * 16
    pltpu.sync_copy(g_hbm, gidx_v)

    for k in range(per_sc):
        c = wid + k * n_workers

        @pl.when(c < n_planes)
        def _do():
            pltpu.sync_copy(in_hbm.at[pl.ds(c * plane_w, plane_w)], stage_v)

            def chunk_body(m, _):
                gvec = gidx_v[pl.ds(m * 16, 16)]
                out_stage_v[pl.ds(m * 16, 16)] = plsc.load_gather(
                    stage_v, [gvec])
                return ()

            lax.fori_loop(0, chunks, chunk_body, ())
            pltpu.sync_copy(out_stage_v, out_hbm.at[pl.ds(c * out_w, out_w)])


def kernel(pose):
    frames = pose.shape[0]
    tail = tuple(pose.shape[1:])
    if frames < _MAX_LEN:
        pad = jnp.zeros((_MAX_LEN - frames,) + tail, dtype=pose.dtype)
        return jnp.concatenate([pose, pad], axis=0)

    chunk = frames // _MAX_LEN
    offs = jax.random.randint(jax.random.key(42), (_MAX_LEN,), 0, chunk)
    idx = jnp.arange(_MAX_LEN, dtype=jnp.int32) * chunk + offs.astype(jnp.int32)

    npersons, nkp, ndim = tail
    n_planes = nkp * ndim
    tiles_in = frames // _LANE
    tiles_out = _MAX_LEN // _LANE
    plane_w = tiles_in * npersons * _LANE
    out_w = tiles_out * npersons * _LANE

    q = np.arange(out_w)
    j = np.asarray((q // (npersons * _LANE)) * _LANE + (q % _LANE))
    p = np.asarray((q // _LANE) % npersons, dtype=np.int32)
    i = idx[j]
    g_const = ((i // _LANE) * (npersons * _LANE) + jnp.asarray(p * _LANE)
               + (i % _LANE)).astype(jnp.int32)

    in_flat = (pose.reshape(tiles_in, _LANE, npersons, nkp, ndim)
               .transpose(3, 4, 0, 2, 1)
               .reshape(n_planes * plane_w))

    info = plsc.get_sparse_core_info()
    n_workers = info.num_cores * info.num_subcores
    per_sc = -(-n_planes // n_workers)
    mesh = plsc.VectorSubcoreMesh(core_axis_name="c", subcore_axis_name="s")

    out_flat = pl.kernel(
        functools.partial(_select_body, info.num_cores, n_planes, plane_w,
                          out_w, per_sc, out_w // 16),
        mesh=mesh,
        compiler_params=pltpu.CompilerParams(needs_layout_passes=False),
        out_type=jax.ShapeDtypeStruct((n_planes * out_w,), jnp.float32),
        scratch_types=[
            pltpu.VMEM((plane_w,), jnp.float32),
            pltpu.VMEM((out_w,), jnp.int32),
            pltpu.VMEM((out_w,), jnp.float32),
        ],
    )(in_flat, g_const)

    out = (out_flat.reshape(nkp, ndim, tiles_out, npersons, _LANE)
           .transpose(2, 4, 3, 0, 1)
           .reshape(_MAX_LEN, npersons, nkp, ndim))
    return out[:, None]


# speedup vs baseline: 1.2990x; 1.2990x over previous
import functools

import jax
import jax.numpy as jnp
import numpy as np
from jax import lax
from jax.experimental import pallas as pl
from jax.experimental.pallas import tpu as pltpu
from jax.experimental.pallas import tpu_sc as plsc

_MAX_LEN = 512
_LANE = 128

_OFFS_DIGITS = (
    "42715317620213423237634341034754563462157475104435432307323116003140"
    "21453544621170541000067560317721145474621137341350633023246614722576"
    "12117434263311723003613264636377077441676023456776504555457303661115"
    "70227117147352314413050444517163047550021150022147202507437305622072"
    "06701425553063312573415330777664277033304001457217056640032424441763"
    "41156032617463424724244614061643056426710417070233305730204452215672"
    "35265032126105375777554144520716474145377006547231023034772244246336"
    "140030102670242044401760412031215026"
)


def _select_body(n_cores, n_subcores, n_planes, per_w, plane_w, out_w, chunks,
                 in_hbm, g_hbm, out_hbm, stage_v, gidx_v, out_stage_v,
                 sem0, sem1):
    wid = lax.axis_index("s") * n_cores + lax.axis_index("c")
    n_workers = n_cores * n_subcores
    s = (wid * (n_planes - per_w)) // (n_workers - 1)
    pltpu.sync_copy(g_hbm, gidx_v)

    sems = (sem0, sem1)

    def in_copy(k):
        b = k % 2
        return pltpu.async_copy(
            in_hbm.at[pl.ds((s + k) * plane_w, plane_w)],
            stage_v.at[pl.ds(b * plane_w, plane_w)],
            sems[b])

    pending = in_copy(0)
    for k in range(per_w):
        b = k % 2
        pending.wait()
        if k + 1 < per_w:
            pending = in_copy(k + 1)
        for m in range(chunks):
            gvec = gidx_v[pl.ds(b * out_w + m * 16, 16)]
            out_stage_v[pl.ds(k * out_w + m * 16, 16)] = plsc.load_gather(
                stage_v, [gvec])

    pltpu.sync_copy(out_stage_v, out_hbm.at[pl.ds(s * out_w, per_w * out_w)])


def kernel(pose):
    frames = pose.shape[0]
    tail = tuple(pose.shape[1:])
    if frames < _MAX_LEN:
        pad = jnp.zeros((_MAX_LEN - frames,) + tail, dtype=pose.dtype)
        return jnp.concatenate([pose, pad], axis=0)

    chunk = frames // _MAX_LEN
    npersons, nkp, ndim = tail
    n_planes = nkp * ndim
    tiles_in = frames // _LANE
    tiles_out = _MAX_LEN // _LANE
    plane_w = tiles_in * npersons * _LANE
    out_w = tiles_out * npersons * _LANE

    q = np.arange(out_w)
    j = (q // (npersons * _LANE)) * _LANE + (q % _LANE)
    p = (q // _LANE) % npersons
    if chunk == 8:
        offs = np.array([int(c) for c in _OFFS_DIGITS], dtype=np.int64)
        idx = np.arange(_MAX_LEN, dtype=np.int64) * chunk + offs
        i = idx[j]
        g = (i // _LANE) * (npersons * _LANE) + p * _LANE + (i % _LANE)
        g_const = jnp.asarray(
            np.concatenate([g, g + plane_w]), dtype=jnp.int32)
    else:
        offs = jax.random.randint(jax.random.key(42), (_MAX_LEN,), 0, chunk)
        idx = jnp.arange(_MAX_LEN, dtype=jnp.int32) * chunk + offs.astype(
            jnp.int32)
        i = idx[jnp.asarray(j)]
        g = ((i // _LANE) * (npersons * _LANE) + jnp.asarray(p * _LANE)
             + (i % _LANE))
        g_const = jnp.concatenate([g, g + plane_w]).astype(jnp.int32)

    in_flat = (pose.reshape(tiles_in, _LANE, npersons, nkp, ndim)
               .transpose(3, 4, 0, 2, 1)
               .reshape(n_planes * plane_w))

    info = plsc.get_sparse_core_info()
    n_workers = info.num_cores * info.num_subcores
    per_w = -(-n_planes // n_workers)
    mesh = plsc.VectorSubcoreMesh(core_axis_name="c", subcore_axis_name="s")

    out_flat = pl.kernel(
        functools.partial(_select_body, info.num_cores, info.num_subcores,
                          n_planes, per_w, plane_w, out_w, out_w // 16),
        mesh=mesh,
        compiler_params=pltpu.CompilerParams(needs_layout_passes=False),
        out_type=jax.ShapeDtypeStruct((n_planes * out_w,), jnp.float32),
        scratch_types=[
            pltpu.VMEM((2 * plane_w,), jnp.float32),
            pltpu.VMEM((2 * out_w,), jnp.int32),
            pltpu.VMEM((per_w * out_w,), jnp.float32),
            pltpu.SemaphoreType.DMA,
            pltpu.SemaphoreType.DMA,
        ],
    )(in_flat, g_const)

    out = (out_flat.reshape(nkp, ndim, tiles_out, npersons, _LANE)
           .transpose(2, 4, 3, 0, 1)
           .reshape(_MAX_LEN, npersons, nkp, ndim))
    return out[:, None]
